# Initial kernel scaffold; baseline (speedup 1.0000x reference)
import jax
import jax.numpy as jnp
from jax.experimental import pallas as pl


def kernel(x, edge_index, W1, att_src1, att_dst1, b1, W2, att_src2, att_dst2, b2):
    raise NotImplementedError("write your pallas kernel here")


# speedup vs baseline: 219.7932x; 219.7932x over previous
import functools

import jax
import jax.numpy as jnp
from jax import lax
from jax.experimental import pallas as pl
from jax.experimental.pallas import tpu as pltpu
from jax.experimental.pallas import tpu_sc as plsc

N_NODES = 50000
NPAD = 51200
ZS = NPAD // 16
N_EDGES = 800000
LANES = 16
TILES = 32
CHUNK = 128
CPT = 200
SBLK = 40
EPAD = TILES * CPT * CHUNK
NBLK = 16
BLK = NPAD // NBLK

_mesh = plsc.VectorSubcoreMesh(
    core_axis_name="c", subcore_axis_name="s", num_cores=2, num_subcores=16)


def _leaky(v):
    return jnp.maximum(v, 0.2 * v)


def _make_edge_pass(nheads):
    @functools.partial(
        pl.kernel,
        out_type=jax.ShapeDtypeStruct((2 * NPAD * 16,), jnp.float32),
        mesh=_mesh,
        scratch_types=[
            pltpu.VMEM((50048,), jnp.float32),
            pltpu.VMEM((SBLK, CHUNK), jnp.int32),
            pltpu.VMEM((SBLK, CHUNK), jnp.int32),
            pltpu.VMEM((CHUNK, 16), jnp.float32),
            pltpu.VMEM((16,), jnp.float32),
            pltpu.VMEM((CHUNK * 16,), jnp.float32),
            pltpu.VMEM((CHUNK,), jnp.int32),
            pltpu.VMEM_SHARED((NPAD, 16), jnp.float32),
        ],
        compiler_params=pltpu.CompilerParams(needs_layout_passes=False),
    )
    def edge_pass(x_hbm, src_hbm, dst_hbm, coef_hbm, out_hbm,
                  x_v, src_v, dst_v, rows_v, coef_v, flat_v, zidx_v, acc_sh):
        c = lax.axis_index("c")
        s = lax.axis_index("s")
        wid = c * 16 + s

        pltpu.sync_copy(x_hbm.at[pl.ds(0, 50048)], x_v)
        pltpu.sync_copy(coef_hbm, coef_v)
        iota16 = lax.iota(jnp.int32, LANES)
        zero16 = jnp.zeros((LANES,), jnp.float32)
        for r in range(CHUNK):
            rows_v[r, :] = zero16
        def fill_zidx(base):
            for k in range(CHUNK // LANES):
                zidx_v[pl.ds(k * LANES, LANES)] = base + k * LANES + iota16

        def zero_blk(b, carry):
            fill_zidx(s * ZS + b * CHUNK)
            pltpu.sync_copy(rows_v, acc_sh.at[zidx_v])
            return carry

        lax.fori_loop(0, ZS // CHUNK, zero_blk, 0)
        coef_vec = coef_v[...]
        cs = [coef_vec[h] for h in range(nheads)]
        cd = [coef_vec[8 + h] for h in range(nheads)]
        plsc.subcore_barrier()

        def chunk_body(j, carry):
            def step(k, kc):
                s16 = src_v[j, pl.ds(k * LANES, LANES)]
                d16 = dst_v[j, pl.ds(k * LANES, LANES)]
                xs = plsc.load_gather(x_v, [s16])
                xd = plsc.load_gather(x_v, [d16])
                r_idx = iota16 + k * LANES
                for h in range(nheads):
                    t = jnp.exp(_leaky(xs * cs[h] + xd * cd[h]))
                    plsc.store_scatter(
                        rows_v, [r_idx, jnp.full((LANES,), h, jnp.int32)], t)
                    plsc.store_scatter(
                        rows_v, [r_idx, jnp.full((LANES,), h + 8, jnp.int32)],
                        t * xs)
                return kc

            lax.fori_loop(0, CHUNK // LANES, step, 0)
            pltpu.sync_copy(rows_v, acc_sh.at[dst_v.at[j]], add=True)
            return carry

        for stage in range(CPT // SBLK):
            row0 = wid * CPT + stage * SBLK
            pltpu.sync_copy(src_hbm.at[pl.ds(row0, SBLK)], src_v)
            pltpu.sync_copy(dst_hbm.at[pl.ds(row0, SBLK)], dst_v)
            lax.fori_loop(0, SBLK, chunk_body, 0)
        plsc.subcore_barrier()

        def repack(r, carry):
            flat_v[pl.ds(r * 16, LANES)] = rows_v[r, :]
            return carry

        base = (c * NPAD + s * ZS) * 16

        def epi_blk(b, carry):
            fill_zidx(s * ZS + b * CHUNK)
            pltpu.sync_copy(acc_sh.at[zidx_v], rows_v)
            lax.fori_loop(0, CHUNK, repack, 0)
            pltpu.sync_copy(flat_v,
                            out_hbm.at[pl.ds(base + b * CHUNK * 16, CHUNK * 16)])
            return carry

        lax.fori_loop(0, ZS // CHUNK, epi_blk, 0)

    return edge_pass


_edge_pass1 = _make_edge_pass(8)
_edge_pass2 = _make_edge_pass(1)


def _node_stage1_body(a0_ref, a1_ref, x_ref, csum_ref, w1f_ref, b1_ref,
                      w2_ref, h2_ref):
    xb = x_ref[...]
    ts = jnp.exp(_leaky(xb * csum_ref[...]))
    a = a0_ref[...] + a1_ref[...]
    den = a[:, :8] + ts
    num = a[:, 8:] + ts * xb
    sval = num / (den + 1e-16)
    srep = jnp.concatenate(
        [jnp.broadcast_to(sval[:, h:h + 1], (BLK, 16)) for h in range(8)],
        axis=1)
    o = srep * w1f_ref[...] + b1_ref[...]
    z = jnp.where(o > 0, o, jnp.exp(jnp.minimum(o, 0.0)) - 1.0)
    h2_ref[...] = jnp.sum(z * w2_ref[...], axis=1, keepdims=True)


_node_stage1 = pl.pallas_call(
    _node_stage1_body,
    grid=(NBLK,),
    in_specs=[
        pl.BlockSpec((BLK, 16), lambda i: (i, 0)),
        pl.BlockSpec((BLK, 16), lambda i: (i, 0)),
        pl.BlockSpec((BLK, 1), lambda i: (i, 0)),
        pl.BlockSpec((1, 8), lambda i: (0, 0)),
        pl.BlockSpec((1, 128), lambda i: (0, 0)),
        pl.BlockSpec((1, 128), lambda i: (0, 0)),
        pl.BlockSpec((1, 128), lambda i: (0, 0)),
    ],
    out_specs=pl.BlockSpec((BLK, 1), lambda i: (i, 0)),
    out_shape=jax.ShapeDtypeStruct((NPAD, 1), jnp.float32),
)


def _node_stage2_body(a0_ref, a1_ref, h2_ref, k_ref, out_ref):
    h2 = h2_ref[...]
    ts = jnp.exp(_leaky(h2 * k_ref[0, 0]))
    a = a0_ref[...] + a1_ref[...]
    den = a[:, 0:1] + ts
    num = a[:, 8:9] + ts * h2
    out_ref[...] = num / (den + 1e-16) + k_ref[0, 1]


_node_stage2 = pl.pallas_call(
    _node_stage2_body,
    grid=(NBLK,),
    in_specs=[
        pl.BlockSpec((BLK, 16), lambda i: (i, 0)),
        pl.BlockSpec((BLK, 16), lambda i: (i, 0)),
        pl.BlockSpec((BLK, 1), lambda i: (i, 0)),
        pl.BlockSpec((1, 2), lambda i: (0, 0)),
    ],
    out_specs=pl.BlockSpec((BLK, 1), lambda i: (i, 0)),
    out_shape=jax.ShapeDtypeStruct((NPAD, 1), jnp.float32),
)


@jax.jit
def kernel(x, edge_index, W1, att_src1, att_dst1, b1, W2, att_src2, att_dst2, b2):
    src = edge_index[0].astype(jnp.int32)
    dst = edge_index[1].astype(jnp.int32)
    pad = EPAD - N_EDGES
    src3d = jnp.pad(src, (0, pad)).reshape(TILES * CPT, CHUNK)
    dst3d = jnp.pad(dst, (0, pad), constant_values=N_NODES).reshape(
        TILES * CPT, CHUNK)
    xf = jnp.pad(x[:, 0], (0, NPAD - N_NODES))

    W1r = W1.reshape(8, 16)
    c_src = (W1r * att_src1[0]).sum(-1)
    c_dst = (W1r * att_dst1[0]).sum(-1)
    coef1 = jnp.concatenate([c_src, c_dst])
    acc1 = _edge_pass1(xf, src3d, dst3d, coef1).reshape(2, NPAD, 16)

    csum = (c_src + c_dst).reshape(1, 8)
    h2 = _node_stage1(acc1[0], acc1[1], xf.reshape(NPAD, 1), csum,
                      W1.reshape(1, 128), b1.reshape(1, 128),
                      W2.reshape(1, 128))

    a_s2 = att_src2[0, 0, 0]
    a_d2 = att_dst2[0, 0, 0]
    coef2 = jnp.zeros((16,), jnp.float32).at[0].set(a_s2).at[8].set(a_d2)
    acc2 = _edge_pass2(h2[:, 0], src3d, dst3d, coef2).reshape(2, NPAD, 16)

    k2 = jnp.stack([a_s2 + a_d2, b2[0]]).reshape(1, 2)
    out = _node_stage2(acc2[0], acc2[1], h2, k2)
    return out[:N_NODES]
